# baseline (speedup 1.0000x reference)
import jax
import jax.numpy as jnp
from jax.experimental import pallas as pl


BATCH = 64
NUM_NODES = 10000
EMB_DIM = 64
NP2 = NUM_NODES // 2

B_TILE = 16
P_TILE = 1024


def _body(v_ref, ez_ref, ep_ref, out_ref):
    ez = ez_ref[...][None, :, :]
    out_ref[...] = jnp.broadcast_to(ez, out_ref.shape)


def kernel(node_values, emb_neg, emb_zero, emb_pos):
    ez2 = emb_zero.reshape(NP2, 2 * EMB_DIM)
    ep2 = emb_pos.reshape(NP2, 2 * EMB_DIM)
    grid = (pl.cdiv(NP2, P_TILE), BATCH // B_TILE)
    out = pl.pallas_call(
        _body,
        grid=grid,
        in_specs=[
            pl.BlockSpec((B_TILE, 2 * P_TILE), lambda n, b: (b, n)),
            pl.BlockSpec((P_TILE, 2 * EMB_DIM), lambda n, b: (n, 0)),
            pl.BlockSpec((P_TILE, 2 * EMB_DIM), lambda n, b: (n, 0)),
        ],
        out_specs=pl.BlockSpec((B_TILE, P_TILE, 2 * EMB_DIM), lambda n, b: (b, n, 0)),
        out_shape=jax.ShapeDtypeStruct((BATCH, NP2, 2 * EMB_DIM), jnp.float32),
    )(node_values, ez2, ep2)
    return out.reshape(BATCH, NUM_NODES, EMB_DIM)
